# baseline (speedup 1.0000x reference)
import jax
import jax.numpy as jnp
from jax.experimental import pallas as pl
from jax.experimental.pallas import tpu as pltpu

_N = 10000
_G = 64
_D = 48


def _fused_body(x_hbm, b_ref, g_ref, be_ref, w2a_ref, b2a_ref, w2b_ref, b2b_ref,
                w2c_ref, b2c_ref, w2d_ref, b2d_ref, wo_ref, bo_ref, out_ref,
                x_vmem, dma_sem):
    cp = pltpu.make_async_copy(x_hbm, x_vmem, dma_sem)
    cp.start()
    cp.wait()
    x = x_vmem[:, :]
    n = jnp.float32(_N)

    colsum = jnp.sum(x, axis=0, keepdims=True)
    colsq = jnp.sum(x * x, axis=0, keepdims=True)
    mu = colsum / n
    var = colsq / n - mu * mu
    inv = jax.lax.rsqrt(var + 1e-5)

    bat = b_ref[:, :]
    onehot_t = (bat == jax.lax.broadcasted_iota(jnp.int32, (_G, 1), 0)
                ).astype(jnp.float32)
    sums = jax.lax.dot_general(onehot_t, x, (((1,), (0,)), ((), ())),
                               preferred_element_type=jnp.float32)
    cnt = jnp.sum(onehot_t, axis=1, keepdims=True)

    gamma = g_ref[:]
    beta = be_ref[:]
    seg_bn = (sums - cnt * mu) * inv * gamma + cnt * beta
    x3 = seg_bn / jnp.maximum(cnt, 1.0)

    def mm(a, w_ref, bias_ref):
        return jax.lax.dot_general(
            a, w_ref[:, :], (((1,), (0,)), ((), ())),
            preferred_element_type=jnp.float32) + bias_ref[:]

    h = jnp.maximum(mm(x3, w2a_ref, b2a_ref), 0.0)
    h = jnp.maximum(mm(h, w2b_ref, b2b_ref), 0.0)
    h = jnp.maximum(mm(h, w2c_ref, b2c_ref), 0.0)
    h = mm(h, w2d_ref, b2d_ref)
    out_ref[:, :] = mm(h, wo_ref, bo_ref)


@jax.jit
def _fused(X, batch_row, bn_gamma, bn_beta, W2a, b2a, W2b, b2b, W2c, b2c,
           W2d, b2d, Wo, bo):
    n_in = 14
    return pl.pallas_call(
        _fused_body,
        out_shape=jax.ShapeDtypeStruct((_G, 1), jnp.float32),
        in_specs=[pl.BlockSpec(memory_space=pl.ANY)]
        + [pl.BlockSpec(memory_space=pltpu.VMEM)] * (n_in - 1),
        scratch_shapes=[pltpu.VMEM((_N, _D), jnp.float32),
                        pltpu.SemaphoreType.DMA],
    )(X, batch_row, bn_gamma, bn_beta, W2a, b2a, W2b, b2b, W2c, b2c,
      W2d, b2d, Wo, bo)


def kernel(X, edge_index, batch, bn_gamma, bn_beta, W1a, b1a, W1b, b1b,
           W1c, b1c, Wc1, bc1, cn_gamma, cn_beta, Wc2, bc2, t,
           W2a, b2a, W2b, b2b, W2c, b2c, W2d, b2d, Wo, bo):
    return _fused(
        X,
        batch.reshape(1, _N),
        bn_gamma, bn_beta,
        W2a, b2a, W2b, b2b, W2c, b2c, W2d, b2d, Wo, bo,
    )


# speedup vs baseline: 1.9216x; 1.9216x over previous
import jax
import jax.numpy as jnp
from jax.experimental import pallas as pl

_N = 10000
_G = 64
_D = 48


def _fused_body(xt_ref, b_ref, g_ref, be_ref, w2a_ref, b2a_ref, w2b_ref,
                b2b_ref, w2c_ref, b2c_ref, w2d_ref, b2d_ref, wo_ref, bo_ref,
                out_ref):
    xt = xt_ref[:, :]
    n = jnp.float32(_N)

    colsum = jnp.sum(xt, axis=1)
    colsq = jnp.sum(xt * xt, axis=1)
    mu = colsum / n
    var = colsq / n - mu * mu
    inv = jax.lax.rsqrt(var + 1e-5)

    bat = b_ref[:, :]
    onehot_t = (bat == jax.lax.broadcasted_iota(jnp.int32, (_G, 1), 0)
                ).astype(jnp.float32)
    sums = jax.lax.dot_general(onehot_t, xt, (((1,), (1,)), ((), ())),
                               preferred_element_type=jnp.float32)
    cnt = jnp.sum(onehot_t, axis=1, keepdims=True)

    gamma = g_ref[:]
    beta = be_ref[:]
    seg_bn = (sums - cnt * mu) * (inv * gamma) + cnt * beta
    x3 = seg_bn / jnp.maximum(cnt, 1.0)

    def mm(a, w_ref, bias_ref):
        return jax.lax.dot_general(
            a, w_ref[:, :], (((1,), (0,)), ((), ())),
            preferred_element_type=jnp.float32) + bias_ref[:]

    h = jnp.maximum(mm(x3, w2a_ref, b2a_ref), 0.0)
    h = jnp.maximum(mm(h, w2b_ref, b2b_ref), 0.0)
    h = jnp.maximum(mm(h, w2c_ref, b2c_ref), 0.0)
    h = mm(h, w2d_ref, b2d_ref)
    out_ref[:, :] = mm(h, wo_ref, bo_ref)


@jax.jit
def _fused(XT, batch_row, bn_gamma, bn_beta, W2a, b2a, W2b, b2b, W2c, b2c,
           W2d, b2d, Wo, bo):
    return pl.pallas_call(
        _fused_body,
        out_shape=jax.ShapeDtypeStruct((_G, 1), jnp.float32),
    )(XT, batch_row, bn_gamma, bn_beta, W2a, b2a, W2b, b2b, W2c, b2c,
      W2d, b2d, Wo, bo)


def kernel(X, edge_index, batch, bn_gamma, bn_beta, W1a, b1a, W1b, b1b,
           W1c, b1c, Wc1, bc1, cn_gamma, cn_beta, Wc2, bc2, t,
           W2a, b2a, W2b, b2b, W2c, b2c, W2d, b2d, Wo, bo):
    return _fused(
        X.T,
        batch.reshape(1, _N),
        bn_gamma, bn_beta,
        W2a, b2a, W2b, b2b, W2c, b2c, W2d, b2d, Wo, bo,
    )


# speedup vs baseline: 3.9722x; 2.0672x over previous
import jax
import jax.numpy as jnp
from jax.experimental import pallas as pl

_N = 10000
_G = 64
_D = 48


def _fused_body(xt_ref, b_ref, g_ref, be_ref, w2a_ref, b2a_ref, w2b_ref,
                b2b_ref, w2c_ref, b2c_ref, w2d_ref, b2d_ref, wo_row_ref,
                bo_ref, out_ref):
    xt = xt_ref[:, :]
    n = jnp.float32(_N)

    colsum = jnp.sum(xt, axis=1)
    colsq = jnp.sum(xt * xt, axis=1)
    mu = colsum / n
    var = colsq / n - mu * mu
    inv = jax.lax.rsqrt(var + 1e-5)

    bat = b_ref[:].reshape(1, _N)
    onehot_t = (bat == jax.lax.broadcasted_iota(jnp.int32, (_G, 1), 0)
                ).astype(jnp.float32)
    sums = jax.lax.dot_general(onehot_t, xt, (((1,), (1,)), ((), ())),
                               preferred_element_type=jnp.float32)
    cnt = jnp.sum(onehot_t, axis=1, keepdims=True)

    gamma = g_ref[:]
    beta = be_ref[:]
    seg_bn = (sums - cnt * mu) * (inv * gamma) + cnt * beta
    x3 = seg_bn / jnp.maximum(cnt, 1.0)

    def mm(a, w_ref, bias_ref):
        return jax.lax.dot_general(
            a, w_ref[:, :], (((1,), (0,)), ((), ())),
            preferred_element_type=jnp.float32) + bias_ref[:]

    h = jnp.maximum(mm(x3, w2a_ref, b2a_ref), 0.0)
    h = jnp.maximum(mm(h, w2b_ref, b2b_ref), 0.0)
    h = jnp.maximum(mm(h, w2c_ref, b2c_ref), 0.0)
    h = mm(h, w2d_ref, b2d_ref)
    out_t = jax.lax.dot_general(h, wo_row_ref[:, :], (((1,), (1,)), ((), ())),
                                preferred_element_type=jnp.float32)
    out_ref[:, :] = out_t.reshape(1, _G) + bo_ref[:]


@jax.jit
def _fused(XT, batch, bn_gamma, bn_beta, W2a, b2a, W2b, b2b, W2c, b2c,
           W2d, b2d, Wo_row, bo):
    out_t = pl.pallas_call(
        _fused_body,
        out_shape=jax.ShapeDtypeStruct((1, _G), jnp.float32),
    )(XT, batch, bn_gamma, bn_beta, W2a, b2a, W2b, b2b, W2c, b2c,
      W2d, b2d, Wo_row, bo)
    return out_t.reshape(_G, 1)


def kernel(X, edge_index, batch, bn_gamma, bn_beta, W1a, b1a, W1b, b1b,
           W1c, b1c, Wc1, bc1, cn_gamma, cn_beta, Wc2, bc2, t,
           W2a, b2a, W2b, b2b, W2c, b2c, W2d, b2d, Wo, bo):
    return _fused(
        X.T,
        batch,
        bn_gamma, bn_beta,
        W2a, b2a, W2b, b2b, W2c, b2c, W2d, b2d, Wo.T, bo,
    )
